# baseline (speedup 1.0000x reference)
import functools

import jax
import jax.numpy as jnp
from jax import lax
from jax.experimental import pallas as pl
from jax.experimental.pallas import tpu as pltpu
from jax.experimental.pallas import tpu_sc as plsc

_B = 1024
_S = 50
_D = 1000
_DPAD = 1024
_H = 8
_W = _DPAD // _H
_NU = _S * (_DPAD // 128)


@functools.lru_cache(maxsize=None)
def _make_tgather():
    info = plsc.get_sparse_core_info()
    nw = info.num_cores * info.num_subcores
    mesh = plsc.VectorSubcoreMesh(core_axis_name="c", subcore_axis_name="s")

    @functools.partial(
        pl.kernel,
        mesh=mesh,
        out_type=jax.ShapeDtypeStruct((_S, _D, _B), jnp.float32),
        compiler_params=pltpu.CompilerParams(needs_layout_passes=False),
        scratch_types=(
            [pltpu.VMEM((_H * 128,), jnp.int32)]
            + [pltpu.VMEM((128, _W), jnp.float32)] * 2
            + [pltpu.VMEM((_W, 128), jnp.float32)] * 2
            + [pltpu.SemaphoreType.DMA] * 4
        ),
    )
    def tgather_kernel(idx_hbm, table_hbm, out_hbm,
                       idx_u, g0, g1, t0, t1, gs0, gs1, os0, os1):
        wid = lax.axis_index("s") * info.num_cores + lax.axis_index("c")
        gbufs, tbufs = (g0, g1), (t0, t1)
        gsems, osems = (gs0, gs1), (os0, os1)
        n_units = jnp.where(wid < _NU % nw, _NU // nw + 1, _NU // nw)

        lanes = lax.iota(jnp.int32, 16)

        idx_bs = [lanes + bg * 16 for bg in range(8)]

        def transpose_block(gb, tb_buf, width):
            @plsc.parallel_loop(0, width, unroll=8)
            def _(vv):
                idx_v = lax.broadcast_in_dim(vv, (16,), ())
                for bg in range(8):
                    vals = plsc.load_gather(gb, [idx_bs[bg], idx_v])
                    tb_buf[vv, pl.ds(bg * 16, 16)] = vals

        def unit_body(k, _):
            u = wid + k * nw
            s = u // 8
            tb = u % 8
            pltpu.sync_copy(
                idx_hbm.at[pl.ds(pl.multiple_of(u * (_H * 128), 128),
                                 _H * 128)],
                idx_u)

            gh = [None] * _H
            oh = [None] * _H

            def start_gather(h):
                return pltpu.async_copy(
                    table_hbm.at[idx_u.at[pl.ds(h * 128, 128)]],
                    gbufs[h % 2], gsems[h % 2])

            gh[0] = start_gather(0)
            for h in range(_H):
                if h + 1 < _H:
                    gh[h + 1] = start_gather(h + 1)
                gh[h].wait()
                if h >= 2:
                    oh[h - 2].wait()
                width = min(_W, _D - h * _W)
                transpose_block(gbufs[h % 2], tbufs[h % 2], width)
                oh[h] = pltpu.async_copy(
                    tbufs[h % 2].at[pl.ds(0, width), :],
                    out_hbm.at[s, pl.ds(h * _W, width),
                               pl.ds(pl.multiple_of(tb * 128, 128), 128)],
                    osems[h % 2])
            oh[_H - 2].wait()
            oh[_H - 1].wait()
            return _

        lax.fori_loop(0, n_units, unit_body, 0, unroll=False)

    return tgather_kernel


def kernel(idx, table):
    b, s = idx.shape
    idx_t = idx.T.reshape(_S, _H, 128).astype(jnp.int32)
    idx4 = (idx_t[:, :, None, :]
            + (jnp.arange(_H, dtype=jnp.int32) * _D)[None, None, :, None]
            ).reshape(-1)
    table_h = (jnp.pad(table, ((0, 0), (0, _DPAD - _D)))
               .reshape(_D, _H, _W).transpose(1, 0, 2).reshape(_H * _D, _W))
    out = _make_tgather()(idx4, table_h)
    return out.transpose(2, 0, 1)


# speedup vs baseline: 3.4783x; 3.4783x over previous
import functools

import jax
import jax.numpy as jnp
from jax import lax
from jax.experimental import pallas as pl
from jax.experimental.pallas import tpu as pltpu
from jax.experimental.pallas import tpu_sc as plsc

_B = 1024
_S = 50
_D = 1000
_DPAD = 1024
_H = 8
_W = _DPAD // _H
_NU = _S * (_DPAD // 128)


@functools.lru_cache(maxsize=None)
def _make_tgather():
    info = plsc.get_sparse_core_info()
    nw = info.num_cores * info.num_subcores
    mesh = plsc.VectorSubcoreMesh(core_axis_name="c", subcore_axis_name="s")

    @functools.partial(
        pl.kernel,
        mesh=mesh,
        out_type=jax.ShapeDtypeStruct((_S, _D, _B), jnp.float32),
        compiler_params=pltpu.CompilerParams(needs_layout_passes=False),
        scratch_types=(
            [pltpu.VMEM((_H * 128,), jnp.int32)]
            + [pltpu.VMEM((128, _W), jnp.float32)] * 2
            + [pltpu.VMEM((_W, 128), jnp.float32)] * 2
            + [pltpu.SemaphoreType.DMA] * 4
        ),
    )
    def tgather_kernel(idx_hbm, table_hbm, out_hbm,
                       idx_u, g0, g1, t0, t1, gs0, gs1, os0, os1):
        wid = lax.axis_index("s") * info.num_cores + lax.axis_index("c")
        gbufs, tbufs = (g0, g1), (t0, t1)
        gsems, osems = (gs0, gs1), (os0, os1)
        n_units = jnp.where(wid < _NU % nw, _NU // nw + 1, _NU // nw)

        lanes = lax.iota(jnp.int32, 16)

        idx_bs = [lanes + bg * 16 for bg in range(8)]
        diags = [jnp.bitwise_and(lanes + k, 15) for k in range(16)]

        def transpose_block(gb, tb_buf):
            @plsc.parallel_loop(0, 128, unroll=2)
            def _(i):
                k16 = lax.broadcast_in_dim(i, (16,), ())
                v0 = lax.broadcast_in_dim(jnp.bitwise_and(i, -16), (16,), ())
                idx_col = jnp.bitwise_and(lanes + k16, 15) + v0
                for bg in range(8):
                    vals = plsc.load_gather(gb, [idx_bs[bg], idx_col])
                    plsc.store_scatter(tb_buf, [idx_col, idx_bs[bg]], vals)

        def unit_body(k, _):
            u = wid + k * nw
            s = u // 8
            tb = u % 8
            pltpu.sync_copy(
                idx_hbm.at[pl.ds(pl.multiple_of(u * (_H * 128), 128),
                                 _H * 128)],
                idx_u)

            gh = [None] * _H
            oh = [None] * _H

            def start_gather(h):
                return pltpu.async_copy(
                    table_hbm.at[idx_u.at[pl.ds(h * 128, 128)]],
                    gbufs[h % 2], gsems[h % 2])

            gh[0] = start_gather(0)
            for h in range(_H):
                if h + 1 < _H:
                    gh[h + 1] = start_gather(h + 1)
                gh[h].wait()
                if h >= 2:
                    oh[h - 2].wait()
                width = min(_W, _D - h * _W)
                transpose_block(gbufs[h % 2], tbufs[h % 2])
                oh[h] = pltpu.async_copy(
                    tbufs[h % 2].at[pl.ds(0, width), :],
                    out_hbm.at[s, pl.ds(h * _W, width),
                               pl.ds(pl.multiple_of(tb * 128, 128), 128)],
                    osems[h % 2])
            oh[_H - 2].wait()
            oh[_H - 1].wait()
            return _

        lax.fori_loop(0, n_units, unit_body, 0, unroll=False)

    return tgather_kernel


def kernel(idx, table):
    b, s = idx.shape
    idx_t = idx.T.reshape(_S, _H, 128).astype(jnp.int32)
    idx4 = (idx_t[:, :, None, :]
            + (jnp.arange(_H, dtype=jnp.int32) * _D)[None, None, :, None]
            ).reshape(-1)
    table_h = (jnp.pad(table, ((0, 0), (0, _DPAD - _D)))
               .reshape(_D, _H, _W).transpose(1, 0, 2).reshape(_H * _D, _W))
    out = _make_tgather()(idx4, table_h)
    return out.transpose(2, 0, 1)


# speedup vs baseline: 3.4824x; 1.0012x over previous
import functools

import jax
import jax.numpy as jnp
from jax import lax
from jax.experimental import pallas as pl
from jax.experimental.pallas import tpu as pltpu
from jax.experimental.pallas import tpu_sc as plsc

_B = 1024
_S = 50
_D = 1000
_DPAD = 1024
_H = 8
_W = _DPAD // _H
_NU = _S * (_DPAD // 128)


@functools.lru_cache(maxsize=None)
def _make_tgather():
    info = plsc.get_sparse_core_info()
    nw = info.num_cores * info.num_subcores
    mesh = plsc.VectorSubcoreMesh(core_axis_name="c", subcore_axis_name="s")

    @functools.partial(
        pl.kernel,
        mesh=mesh,
        out_type=jax.ShapeDtypeStruct((_S, _D, _B), jnp.float32),
        compiler_params=pltpu.CompilerParams(needs_layout_passes=False),
        scratch_types=(
            [pltpu.VMEM((_H * 128,), jnp.int32)]
            + [pltpu.VMEM((128, _W), jnp.float32)] * 2
            + [pltpu.VMEM((_W, 128), jnp.float32)] * 2
            + [pltpu.SemaphoreType.DMA] * 4
        ),
    )
    def tgather_kernel(idx_hbm, table_hbm, out_hbm,
                       idx_u, g0, g1, t0, t1, gs0, gs1, os0, os1):
        wid = lax.axis_index("s") * info.num_cores + lax.axis_index("c")
        gbufs, tbufs = (g0, g1), (t0, t1)
        gsems, osems = (gs0, gs1), (os0, os1)
        n_units = jnp.where(wid < _NU % nw, _NU // nw + 1, _NU // nw)

        lanes = lax.iota(jnp.int32, 16)

        idx_bs = [lanes + bg * 16 for bg in range(8)]

        def transpose_block(gb, tb_buf):
            @plsc.parallel_loop(0, 128, unroll=2)
            def _(i):
                k16 = lax.broadcast_in_dim(i, (16,), ())
                v0 = lax.broadcast_in_dim(jnp.bitwise_and(i, -16), (16,), ())
                idx_col = jnp.bitwise_and(lanes + k16, 15) + v0
                for bg in range(8):
                    vals = plsc.load_gather(gb, [idx_bs[bg], idx_col])
                    plsc.store_scatter(tb_buf, [idx_col, idx_bs[bg]], vals)

        def unit_body(k, _):
            u = wid + k * nw
            s = u // 8
            tb = u % 8
            pltpu.sync_copy(
                idx_hbm.at[pl.ds(pl.multiple_of(u * (_H * 128), 128),
                                 _H * 128)],
                idx_u)

            gh = [None] * _H
            oh = [None] * _H

            def start_gather(h):
                return pltpu.async_copy(
                    table_hbm.at[idx_u.at[pl.ds(h * 128, 128)]],
                    gbufs[h % 2], gsems[h % 2])

            gh[0] = start_gather(0)
            for h in range(_H):
                if h + 1 < _H:
                    gh[h + 1] = start_gather(h + 1)
                gh[h].wait()
                if h >= 2:
                    oh[h - 2].wait()
                width = min(_W, _D - h * _W)
                transpose_block(gbufs[h % 2], tbufs[h % 2])
                oh[h] = pltpu.async_copy(
                    tbufs[h % 2].at[pl.ds(0, width), :],
                    out_hbm.at[s, pl.ds(h * _W, width),
                               pl.ds(pl.multiple_of(tb * 128, 128), 128)],
                    osems[h % 2])
            oh[_H - 2].wait()
            oh[_H - 1].wait()
            return _

        lax.fori_loop(0, n_units, unit_body, 0, unroll=False)

    return tgather_kernel


def kernel(idx, table):
    b, s = idx.shape
    idx_t = idx.T.reshape(_S, _H, 128).astype(jnp.int32)
    idx4 = (idx_t[:, :, None, :]
            + (jnp.arange(_H, dtype=jnp.int32) * _D)[None, None, :, None]
            ).reshape(-1)
    table_h = (jnp.pad(table, ((0, 0), (0, _DPAD - _D)))
               .reshape(_D, _H, _W).transpose(1, 0, 2).reshape(_H * _D, _W))
    out = _make_tgather()(idx4, table_h)
    return out.transpose(2, 0, 1)
